# Initial kernel scaffold; baseline (speedup 1.0000x reference)
import functools

import jax
import jax.numpy as jnp
from jax import lax
from jax.experimental import pallas as pl
from jax.experimental.pallas import tpu as pltpu
from jax.experimental.pallas import tpu_sc as plsc

_VOCAB = 100000
_N_PREFIX = 100
_DIM = 64
_LANES = 16
_NC = 2
_NS = 16
_NW = _NC * _NS
_CHUNK = 512
_SUB = 128


@functools.cache
def _make_gather(n_idx):
    n_per_w = n_idx // _NW
    n_chunks = n_per_w // _CHUNK
    mesh = plsc.VectorSubcoreMesh(core_axis_name="c", subcore_axis_name="s")

    @functools.partial(
        pl.kernel,
        out_type=jax.ShapeDtypeStruct((n_idx, _DIM), jnp.float32),
        mesh=mesh,
        scratch_types=[
            pltpu.VMEM((_CHUNK,), jnp.int32),
            pltpu.VMEM((_CHUNK,), jnp.int32),
            pltpu.VMEM((_CHUNK, _DIM), jnp.float32),
            pltpu.VMEM((_N_PREFIX, _DIM), jnp.float32),
            pltpu.SemaphoreType.DMA,
        ],
    )
    def gather_kernel(w_hbm, wn_hbm, idx_hbm, out_hbm,
                      idxo_v, idxc_v, rows_v, wn_v, sem):
        wid = lax.axis_index("s") * _NC + lax.axis_index("c")
        pltpu.sync_copy(wn_hbm, wn_v)

        def chunk_body(g, carry):
            base = wid * n_per_w + g * _CHUNK
            pltpu.sync_copy(idx_hbm.at[pl.ds(base, _CHUNK)], idxo_v)

            def clamp_body(j, cnt):
                v = idxo_v[pl.ds(j * _LANES, _LANES)]
                m = v >= _VOCAB
                idxc_v[pl.ds(j * _LANES, _LANES)] = jnp.where(m, _VOCAB - 1, v)
                return cnt + jnp.sum(m.astype(jnp.int32))

            cnt = lax.fori_loop(0, _CHUNK // _LANES, clamp_body, jnp.int32(0))

            copies = [
                pltpu.async_copy(
                    w_hbm.at[idxc_v.at[pl.ds(k * _SUB, _SUB)]],
                    rows_v.at[pl.ds(k * _SUB, _SUB)],
                    sem,
                )
                for k in range(_CHUNK // _SUB)
            ]
            for c in copies:
                c.wait()

            @pl.when(cnt > 0)
            def _patch():
                def patch_slice(j, _):
                    v = idxo_v[pl.ds(j * _LANES, _LANES)]
                    m = v >= _VOCAB

                    @pl.when(jnp.sum(m.astype(jnp.int32)) > 0)
                    def _():
                        e = jnp.where(m, v - _VOCAB, 0)
                        rows16 = j * _LANES + lax.iota(jnp.int32, _LANES)

                        def col_body(c, cc):
                            colv = jnp.full((_LANES,), c, jnp.int32)
                            vals = plsc.load_gather(wn_v, [e, colv], mask=m)
                            plsc.store_scatter(rows_v, [rows16, colv], vals,
                                               mask=m)
                            return cc

                        lax.fori_loop(0, _DIM, col_body, jnp.int32(0))

                    return _

                lax.fori_loop(0, _CHUNK // _LANES, patch_slice, jnp.int32(0))

            pltpu.sync_copy(rows_v, out_hbm.at[pl.ds(base, _CHUNK)])
            return carry

        lax.fori_loop(0, n_chunks, chunk_body, jnp.int32(0))

    return gather_kernel


def kernel(input, W, W_new):
    idx = input.reshape(-1).astype(jnp.int32)
    out = _make_gather(idx.shape[0])(W, W_new, idx)
    return out.reshape(*input.shape, _DIM)


# speedup vs baseline: 5.5633x; 5.5633x over previous
import functools

import jax
import jax.numpy as jnp
from jax import lax
from jax.experimental import pallas as pl
from jax.experimental.pallas import tpu as pltpu
from jax.experimental.pallas import tpu_sc as plsc

_VOCAB = 100000
_N_PREFIX = 100
_DIM = 64
_LANES = 16
_NC = 2
_NS = 16
_NW = _NC * _NS
_CHUNK = 512
_SUB = 128


@functools.cache
def _make_gather(n_idx):
    n_per_w = n_idx // _NW
    n_chunks = n_per_w // _CHUNK
    mesh = plsc.VectorSubcoreMesh(core_axis_name="c", subcore_axis_name="s",
                                  num_cores=_NC, num_subcores=_NS)

    @functools.partial(
        pl.kernel,
        out_type=jax.ShapeDtypeStruct((n_idx, _DIM), jnp.float32),
        mesh=mesh,
        compiler_params=pltpu.CompilerParams(use_tc_tiling_on_sc=False,
                                             needs_layout_passes=False),
        scratch_types=[
            pltpu.VMEM((_CHUNK,), jnp.int32),
            pltpu.VMEM((_CHUNK,), jnp.int32),
            pltpu.VMEM((_CHUNK, _DIM), jnp.float32),
            pltpu.VMEM((_N_PREFIX, _DIM), jnp.float32),
            pltpu.SemaphoreType.DMA,
        ],
    )
    def gather_kernel(w_hbm, wn_hbm, idx_hbm, out_hbm,
                      idxo_v, idxc_v, rows_v, wn_v, sem):
        wid = lax.axis_index("s") * _NC + lax.axis_index("c")
        pltpu.sync_copy(wn_hbm, wn_v)

        def chunk_body(g, carry):
            base = wid * n_per_w + g * _CHUNK
            pltpu.sync_copy(idx_hbm.at[pl.ds(base, _CHUNK)], idxo_v)

            def clamp_body(j, has_prefix):
                v = idxo_v[pl.ds(j * _LANES, _LANES)]
                m = v >= _VOCAB
                idxc_v[pl.ds(j * _LANES, _LANES)] = jnp.where(m, _VOCAB - 1, v)
                return has_prefix | jnp.any(m)

            has_prefix = lax.fori_loop(0, _CHUNK // _LANES, clamp_body,
                                       jnp.bool_(False))

            copies = [
                pltpu.async_copy(
                    w_hbm.at[idxc_v.at[pl.ds(k * _SUB, _SUB)]],
                    rows_v.at[pl.ds(k * _SUB, _SUB)],
                    sem,
                )
                for k in range(_CHUNK // _SUB)
            ]
            for c in copies:
                c.wait()

            @pl.when(has_prefix)
            def _patch():
                def patch_slice(j, acc):
                    v = idxo_v[pl.ds(j * _LANES, _LANES)]
                    m = v >= _VOCAB

                    @pl.when(jnp.any(m))
                    def _do_patch():
                        e = jnp.where(m, v - _VOCAB, 0)
                        rows16 = j * _LANES + lax.iota(jnp.int32, _LANES)

                        def col_body(c, cc):
                            colv = jnp.full((_LANES,), c, jnp.int32)
                            vals = plsc.load_gather(wn_v, [e, colv], mask=m)
                            plsc.store_scatter(rows_v, [rows16, colv], vals,
                                               mask=m)
                            return cc

                        lax.fori_loop(0, _DIM, col_body, jnp.int32(0))

                    return acc

                lax.fori_loop(0, _CHUNK // _LANES, patch_slice, jnp.int32(0))

            pltpu.sync_copy(rows_v, out_hbm.at[pl.ds(base, _CHUNK)])
            return carry

        lax.fori_loop(0, n_chunks, chunk_body, jnp.int32(0))

    return gather_kernel


def kernel(input, W, W_new):
    idx = input.reshape(-1).astype(jnp.int32)
    out = _make_gather(idx.shape[0])(W, W_new, idx)
    return out.reshape(*input.shape, _DIM)


# speedup vs baseline: 5.9352x; 1.0668x over previous
import functools

import jax
import jax.numpy as jnp
from jax import lax
from jax.experimental import pallas as pl
from jax.experimental.pallas import tpu as pltpu
from jax.experimental.pallas import tpu_sc as plsc

_VOCAB = 100000
_N_PREFIX = 100
_DIM = 64
_LANES = 16
_NC = 2
_NS = 16
_NW = _NC * _NS
_CHUNK = 512
_SUB = 128
_NBUF = 2


@functools.cache
def _make_gather(n_idx):
    n_per_w = n_idx // _NW
    n_chunks = n_per_w // _CHUNK
    n_steps = n_chunks // _NBUF
    mesh = plsc.VectorSubcoreMesh(core_axis_name="c", subcore_axis_name="s",
                                  num_cores=_NC, num_subcores=_NS)

    @functools.partial(
        pl.kernel,
        out_type=jax.ShapeDtypeStruct((n_idx, _DIM), jnp.float32),
        mesh=mesh,
        compiler_params=pltpu.CompilerParams(use_tc_tiling_on_sc=False,
                                             needs_layout_passes=False),
        scratch_types=[
            pltpu.VMEM((_NBUF, _CHUNK), jnp.int32),
            pltpu.VMEM((_NBUF, _CHUNK), jnp.int32),
            pltpu.VMEM((_NBUF, _CHUNK, _DIM), jnp.float32),
            pltpu.VMEM((_N_PREFIX, _DIM), jnp.float32),
            pltpu.SemaphoreType.DMA,
            pltpu.SemaphoreType.DMA,
            pltpu.SemaphoreType.DMA,
        ],
    )
    def gather_kernel(w_hbm, wn_hbm, idx_hbm, out_hbm,
                      idxo_v, idxc_v, rows_v, wn_v, gsem, ssem0, ssem1):
        wid = lax.axis_index("s") * _NC + lax.axis_index("c")
        wbase = wid * n_per_w
        ssems = (ssem0, ssem1)
        pltpu.sync_copy(wn_hbm, wn_v)

        def do_chunk(g, b, first):
            base = wbase + g * _CHUNK
            idxo = idxo_v.at[b]
            idxc = idxc_v.at[b]
            rows = rows_v.at[b]
            pltpu.sync_copy(idx_hbm.at[pl.ds(base, _CHUNK)], idxo)

            def clamp_body(j, has_prefix):
                v = idxo[pl.ds(j * _LANES, _LANES)]
                m = v >= _VOCAB
                idxc[pl.ds(j * _LANES, _LANES)] = jnp.where(m, _VOCAB - 1, v)
                return has_prefix | jnp.any(m)

            has_prefix = lax.fori_loop(0, _CHUNK // _LANES, clamp_body,
                                       jnp.bool_(False))

            @pl.when(jnp.logical_not(first))
            def _drain_prev():
                pltpu.make_async_copy(
                    rows, out_hbm.at[pl.ds(base, _CHUNK)], ssems[b]).wait()

            copies = [
                pltpu.async_copy(
                    w_hbm.at[idxc.at[pl.ds(k * _SUB, _SUB)]],
                    rows.at[pl.ds(k * _SUB, _SUB)],
                    gsem,
                )
                for k in range(_CHUNK // _SUB)
            ]
            for c in copies:
                c.wait()

            @pl.when(has_prefix)
            def _patch():
                def patch_slice(j, acc):
                    v = idxo[pl.ds(j * _LANES, _LANES)]
                    m = v >= _VOCAB

                    @pl.when(jnp.any(m))
                    def _do_patch():
                        e = jnp.where(m, v - _VOCAB, 0)
                        rows16 = j * _LANES + lax.iota(jnp.int32, _LANES)

                        def col_body(c, cc):
                            colv = jnp.full((_LANES,), c, jnp.int32)
                            vals = plsc.load_gather(wn_v, [e, colv], mask=m)
                            plsc.store_scatter(rows, [rows16, colv], vals,
                                               mask=m)
                            return cc

                        lax.fori_loop(0, _DIM, col_body, jnp.int32(0))

                    return acc

                lax.fori_loop(0, _CHUNK // _LANES, patch_slice, jnp.int32(0))

            pltpu.async_copy(rows, out_hbm.at[pl.ds(base, _CHUNK)], ssems[b])

        def step_body(s, carry):
            for b in range(_NBUF):
                do_chunk(s * _NBUF + b, b, s < 1)
            return carry

        lax.fori_loop(0, n_steps, step_body, jnp.int32(0))

        for b in range(_NBUF):
            g = (n_steps - 1) * _NBUF + b
            pltpu.make_async_copy(
                rows_v.at[b],
                out_hbm.at[pl.ds(wbase + g * _CHUNK, _CHUNK)],
                ssems[b]).wait()

    return gather_kernel


def kernel(input, W, W_new):
    idx = input.reshape(-1).astype(jnp.int32)
    out = _make_gather(idx.shape[0])(W, W_new, idx)
    return out.reshape(*input.shape, _DIM)
